# baseline (speedup 1.0000x reference)
import functools

import jax
import jax.numpy as jnp
from jax import lax
from jax.experimental import pallas as pl
from jax.experimental.pallas import tpu as pltpu
from jax.experimental.pallas import tpu_sc as plsc

_B = 4096
_L = 200
_DIM = 64
_V = 1000000

_NC = 2
_NS = 16
_NW = _NC * _NS


def _gather_kernel(tok_hbm, table_hbm, out_hbm,
                   tokv, rA, rB, oA, oB, gA, gB, sA, sB):
    wid = lax.axis_index("s") * _NC + lax.axis_index("c")

    pltpu.sync_copy(tok_hbm.at[:, wid], tokv)

    iu = [lax.iota(jnp.int32, 16) + 16 * m for m in range(8)]

    def fire(l, rbuf, gsem):
        pltpu.async_copy(
            table_hbm.at[tokv.at[l // 8, l % 8]], rbuf, gsem)

    def wait_g(rbuf, gsem):
        pltpu.make_async_copy(table_hbm.at[pl.ds(0, 128)], rbuf, gsem).wait()

    def shuffle(rbuf, obuf):
        for s in range(8):
            for j in range(8):
                d = 8 * s + j
                fd = jnp.full((16,), d, jnp.int32)
                for m in range(8):
                    v = plsc.load_gather(rbuf, [iu[m], fd])
                    obuf[s, j, pl.ds(16 * m, 16)] = v

    def fire_out(l, obuf, osem):
        pltpu.async_copy(obuf, out_hbm.at[l, :, wid], osem)

    def wait_o(obuf, osem):
        pltpu.make_async_copy(obuf, out_hbm.at[0, :, wid], osem).wait()

    fire(0, rA, gA)
    fire(1, rB, gB)

    def body(p, carry):
        l0 = 2 * p
        wait_g(rA, gA)

        @pl.when(p > 0)
        def _():
            wait_o(oA, sA)

        shuffle(rA, oA)
        fire_out(l0, oA, sA)

        @pl.when(l0 + 2 < _L)
        def _():
            fire(l0 + 2, rA, gA)

        wait_g(rB, gB)

        @pl.when(p > 0)
        def _():
            wait_o(oB, sB)

        shuffle(rB, oB)
        fire_out(l0 + 1, oB, sB)

        @pl.when(l0 + 3 < _L)
        def _():
            fire(l0 + 3, rB, gB)

        return carry

    lax.fori_loop(0, _L // 2, body, 0)
    wait_o(oA, sA)
    wait_o(oB, sB)


def _build():
    mesh = plsc.VectorSubcoreMesh(core_axis_name="c", subcore_axis_name="s")
    return functools.partial(
        pl.kernel,
        mesh=mesh,
        out_type=jax.ShapeDtypeStruct((_L, 8, _NW, 8, 128), jnp.float32),
        scratch_types=[
            pltpu.VMEM((25, 8, 128), jnp.int32),
            pltpu.VMEM((128, _DIM), jnp.float32),
            pltpu.VMEM((128, _DIM), jnp.float32),
            pltpu.VMEM((8, 8, 128), jnp.float32),
            pltpu.VMEM((8, 8, 128), jnp.float32),
            pltpu.SemaphoreType.DMA,
            pltpu.SemaphoreType.DMA,
            pltpu.SemaphoreType.DMA,
            pltpu.SemaphoreType.DMA,
        ],
        compiler_params=pltpu.CompilerParams(
            use_tc_tiling_on_sc=False, needs_layout_passes=False),
    )(_gather_kernel)


_lookup = _build()


@jax.jit
def kernel(token_id, table):
    tok5 = token_id.astype(jnp.int32).T.reshape(25, 8, _NW, 128).transpose(0, 2, 1, 3)
    out5 = _lookup(tok5, table)
    return out5.transpose(2, 4, 0, 1, 3).reshape(_B, _L, _DIM)


# speedup vs baseline: 1.8287x; 1.8287x over previous
import functools

import jax
import jax.numpy as jnp
from jax import lax
from jax.experimental import pallas as pl
from jax.experimental.pallas import tpu as pltpu
from jax.experimental.pallas import tpu_sc as plsc

_B = 4096
_L = 200
_DIM = 64
_V = 1000000

_NC = 2
_NS = 16
_NW = _NC * _NS


def _gather_kernel(tok_hbm, table_hbm, out_hbm,
                   tokv, rA, rB, oA, oB, gA, gB, sA, sB):
    wid = lax.axis_index("s") * _NC + lax.axis_index("c")

    pltpu.sync_copy(tok_hbm.at[:, wid], tokv)

    dks = [lax.iota(jnp.int32, 16) + 16 * k for k in range(4)]
    idx_s = [dk >> 3 for dk in dks]
    idx_j = [dk & 7 for dk in dks]

    def fire(l, rbuf, gsem):
        pltpu.async_copy(
            table_hbm.at[tokv.at[l // 8, l % 8]], rbuf, gsem)

    def wait_g(rbuf, gsem):
        pltpu.make_async_copy(table_hbm.at[pl.ds(0, 128)], rbuf, gsem).wait()

    def shuffle(rbuf, obuf):
        for u in range(128):
            fu = jnp.full((16,), u, jnp.int32)
            for k in range(4):
                v = rbuf[u, pl.ds(16 * k, 16)]
                plsc.store_scatter(obuf, [idx_s[k], idx_j[k], fu], v)

    def fire_out(l, obuf, osem):
        pltpu.async_copy(
            obuf.at[:, :, pl.ds(0, 128)], out_hbm.at[l, :, wid], osem)

    def wait_o(obuf, osem):
        pltpu.make_async_copy(
            obuf.at[:, :, pl.ds(0, 128)], out_hbm.at[0, :, wid], osem
        ).wait()

    fire(0, rA, gA)
    fire(1, rB, gB)

    def body(p, carry):
        l0 = 2 * p
        wait_g(rA, gA)

        @pl.when(p > 0)
        def _():
            wait_o(oA, sA)

        shuffle(rA, oA)
        fire_out(l0, oA, sA)

        @pl.when(l0 + 2 < _L)
        def _():
            fire(l0 + 2, rA, gA)

        wait_g(rB, gB)

        @pl.when(p > 0)
        def _():
            wait_o(oB, sB)

        shuffle(rB, oB)
        fire_out(l0 + 1, oB, sB)

        @pl.when(l0 + 3 < _L)
        def _():
            fire(l0 + 3, rB, gB)

        return carry

    lax.fori_loop(0, _L // 2, body, 0)
    wait_o(oA, sA)
    wait_o(oB, sB)


def _build():
    mesh = plsc.VectorSubcoreMesh(core_axis_name="c", subcore_axis_name="s")
    return functools.partial(
        pl.kernel,
        mesh=mesh,
        out_type=jax.ShapeDtypeStruct((_L, 8, _NW, 8, 128), jnp.float32),
        scratch_types=[
            pltpu.VMEM((25, 8, 128), jnp.int32),
            pltpu.VMEM((128, _DIM), jnp.float32),
            pltpu.VMEM((128, _DIM), jnp.float32),
            pltpu.VMEM((8, 8, 129), jnp.float32),
            pltpu.VMEM((8, 8, 129), jnp.float32),
            pltpu.SemaphoreType.DMA,
            pltpu.SemaphoreType.DMA,
            pltpu.SemaphoreType.DMA,
            pltpu.SemaphoreType.DMA,
        ],
        compiler_params=pltpu.CompilerParams(
            use_tc_tiling_on_sc=False, needs_layout_passes=False),
    )(_gather_kernel)


_lookup = _build()


@jax.jit
def kernel(token_id, table):
    tok5 = token_id.astype(jnp.int32).T.reshape(25, 8, _NW, 128).transpose(0, 2, 1, 3)
    out5 = _lookup(tok5, table)
    return out5.transpose(2, 4, 0, 1, 3).reshape(_B, _L, _DIM)


# speedup vs baseline: 2.0105x; 1.0995x over previous
import functools

import jax
import jax.numpy as jnp
from jax import lax
from jax.experimental import pallas as pl
from jax.experimental.pallas import tpu as pltpu
from jax.experimental.pallas import tpu_sc as plsc

_B = 4096
_L = 200
_DIM = 64
_V = 1000000

_NC = 2
_NS = 16
_NW = _NC * _NS

_DEPTH = 4


def _gather_kernel(tok_hbm, table_hbm, out_hbm,
                   tokv, r0, r1, r2, r3, o0, o1, o2, o3,
                   g0, g1, g2, g3, s0, s1, s2, s3):
    wid = lax.axis_index("s") * _NC + lax.axis_index("c")

    rbs = [r0, r1, r2, r3]
    obs = [o0, o1, o2, o3]
    gsems = [g0, g1, g2, g3]
    osems = [s0, s1, s2, s3]

    pltpu.sync_copy(tok_hbm.at[:, wid], tokv)

    lane = lax.iota(jnp.int32, 16)

    def fire(l, rbuf, gsem):
        pltpu.async_copy(
            table_hbm.at[tokv.at[l // 8, l % 8]], rbuf, gsem)

    def wait_g(rbuf, gsem):
        pltpu.make_async_copy(table_hbm.at[pl.ds(0, 128)], rbuf, gsem).wait()

    def shuffle(rbuf, obuf):
        def body(u, carry):
            fu = jnp.full((16,), 0, jnp.int32) + u
            for k in range(4):
                dk = lane + 16 * k
                v = rbuf[u, pl.ds(16 * k, 16)]
                plsc.store_scatter(obuf, [dk >> 3, dk & 7, fu], v)
            return carry
        lax.fori_loop(0, 128, body, 0)

    def fire_out(l, obuf, osem):
        pltpu.async_copy(
            obuf.at[:, :, pl.ds(0, 128)], out_hbm.at[l, :, wid], osem)

    def wait_o(obuf, osem):
        pltpu.make_async_copy(
            obuf.at[:, :, pl.ds(0, 128)], out_hbm.at[0, :, wid], osem
        ).wait()

    for q in range(_DEPTH):
        fire(q, rbs[q], gsems[q])

    def body(p, carry):
        l0 = _DEPTH * p
        for q in range(_DEPTH):
            l = l0 + q
            wait_g(rbs[q], gsems[q])

            @pl.when(p > 0)
            def _():
                wait_o(obs[q], osems[q])

            shuffle(rbs[q], obs[q])
            fire_out(l, obs[q], osems[q])

            @pl.when(l + _DEPTH < _L)
            def _():
                fire(l + _DEPTH, rbs[q], gsems[q])

        return carry

    lax.fori_loop(0, _L // _DEPTH, body, 0)
    for q in range(_DEPTH):
        wait_o(obs[q], osems[q])


def _build():
    mesh = plsc.VectorSubcoreMesh(core_axis_name="c", subcore_axis_name="s")
    return functools.partial(
        pl.kernel,
        mesh=mesh,
        out_type=jax.ShapeDtypeStruct((_L, 8, _NW, 8, 128), jnp.float32),
        scratch_types=[
            pltpu.VMEM((25, 8, 128), jnp.int32),
            pltpu.VMEM((128, _DIM), jnp.float32),
            pltpu.VMEM((128, _DIM), jnp.float32),
            pltpu.VMEM((128, _DIM), jnp.float32),
            pltpu.VMEM((128, _DIM), jnp.float32),
            pltpu.VMEM((8, 8, 129), jnp.float32),
            pltpu.VMEM((8, 8, 129), jnp.float32),
            pltpu.VMEM((8, 8, 129), jnp.float32),
            pltpu.VMEM((8, 8, 129), jnp.float32),
            pltpu.SemaphoreType.DMA,
            pltpu.SemaphoreType.DMA,
            pltpu.SemaphoreType.DMA,
            pltpu.SemaphoreType.DMA,
            pltpu.SemaphoreType.DMA,
            pltpu.SemaphoreType.DMA,
            pltpu.SemaphoreType.DMA,
            pltpu.SemaphoreType.DMA,
        ],
        compiler_params=pltpu.CompilerParams(
            use_tc_tiling_on_sc=False, needs_layout_passes=False),
    )(_gather_kernel)


_lookup = _build()


@jax.jit
def kernel(token_id, table):
    tok5 = token_id.astype(jnp.int32).T.reshape(25, 8, _NW, 128).transpose(0, 2, 1, 3)
    out5 = _lookup(tok5, table)
    return out5.transpose(2, 4, 0, 1, 3).reshape(_B, _L, _DIM)


# speedup vs baseline: 2.0530x; 1.0212x over previous
import functools

import jax
import jax.numpy as jnp
from jax import lax
from jax.experimental import pallas as pl
from jax.experimental.pallas import tpu as pltpu
from jax.experimental.pallas import tpu_sc as plsc

_B = 4096
_L = 200
_DIM = 64
_V = 1000000

_NC = 2
_NS = 16
_NW = _NC * _NS

_DEPTH = 5


def _gather_kernel(tok_hbm, table_hbm, out_hbm,
                   tokv, r0, r1, r2, r3, r4, o0, o1, o2, o3, o4,
                   g0, g1, g2, g3, g4, s0, s1, s2, s3, s4):
    wid = lax.axis_index("s") * _NC + lax.axis_index("c")

    rbs = [r0, r1, r2, r3, r4]
    obs = [o0, o1, o2, o3, o4]
    gsems = [g0, g1, g2, g3, g4]
    osems = [s0, s1, s2, s3, s4]

    pltpu.sync_copy(tok_hbm.at[:, wid], tokv)

    lane = lax.iota(jnp.int32, 16)
    idx_s = [(lane + 16 * k) >> 3 for k in range(4)]
    idx_j = [(lane + 16 * k) & 7 for k in range(4)]

    def fire(l, rbuf, gsem):
        pltpu.async_copy(
            table_hbm.at[tokv.at[l // 8, l % 8]], rbuf, gsem)

    def wait_g(rbuf, gsem):
        pltpu.make_async_copy(table_hbm.at[pl.ds(0, 128)], rbuf, gsem).wait()

    def shuffle(rbuf, obuf):
        def body(t, carry):
            u0 = 4 * t
            for du in range(4):
                u = u0 + du
                fu = jnp.full((16,), 0, jnp.int32) + u
                for k in range(4):
                    v = rbuf[u, pl.ds(16 * k, 16)]
                    plsc.store_scatter(obuf, [idx_s[k], idx_j[k], fu], v)
            return carry
        lax.fori_loop(0, 32, body, 0)

    def fire_out(l, obuf, osem):
        pltpu.async_copy(
            obuf.at[:, :, pl.ds(0, 128)], out_hbm.at[l, :, wid], osem)

    def wait_o(obuf, osem):
        pltpu.make_async_copy(
            obuf.at[:, :, pl.ds(0, 128)], out_hbm.at[0, :, wid], osem
        ).wait()

    for q in range(_DEPTH):
        fire(q, rbs[q], gsems[q])

    def body(p, carry):
        l0 = _DEPTH * p
        for q in range(_DEPTH):
            l = l0 + q
            wait_g(rbs[q], gsems[q])

            @pl.when(p > 0)
            def _():
                wait_o(obs[q], osems[q])

            shuffle(rbs[q], obs[q])
            fire_out(l, obs[q], osems[q])

            @pl.when(l + _DEPTH < _L)
            def _():
                fire(l + _DEPTH, rbs[q], gsems[q])

        return carry

    lax.fori_loop(0, _L // _DEPTH, body, 0)
    for q in range(_DEPTH):
        wait_o(obs[q], osems[q])


def _build():
    mesh = plsc.VectorSubcoreMesh(core_axis_name="c", subcore_axis_name="s")
    return functools.partial(
        pl.kernel,
        mesh=mesh,
        out_type=jax.ShapeDtypeStruct((_L, 8, _NW, 8, 128), jnp.float32),
        scratch_types=(
            [pltpu.VMEM((25, 8, 128), jnp.int32)]
            + [pltpu.VMEM((128, _DIM), jnp.float32) for _ in range(_DEPTH)]
            + [pltpu.VMEM((8, 8, 129), jnp.float32) for _ in range(_DEPTH)]
            + [pltpu.SemaphoreType.DMA for _ in range(2 * _DEPTH)]
        ),
        compiler_params=pltpu.CompilerParams(
            use_tc_tiling_on_sc=False, needs_layout_passes=False),
    )(_gather_kernel)


_lookup = _build()


@jax.jit
def kernel(token_id, table):
    tok5 = token_id.astype(jnp.int32).T.reshape(25, 8, _NW, 128).transpose(0, 2, 1, 3)
    out5 = _lookup(tok5, table)
    return out5.transpose(2, 4, 0, 1, 3).reshape(_B, _L, _DIM)
